# Initial kernel scaffold; baseline (speedup 1.0000x reference)
import jax
import jax.numpy as jnp
from jax.experimental import pallas as pl


def kernel(feat_f0, feat_f1, feat_c0, feat_c1, coord0_f, coord1_f, b_ids, i_ids, j_ids, image0, W_down, b_down, W_merge, b_merge):
    raise NotImplementedError("write your pallas kernel here")


# speedup vs baseline: 1.8154x; 1.8154x over previous
import functools

import jax
import jax.numpy as jnp
from jax import lax
from jax.experimental import pallas as pl
from jax.experimental.pallas import tpu as pltpu
from jax.experimental.pallas import tpu_sc as plsc

_WIN = 5
_WIN2 = _WIN * _WIN
_LANES = 16


def _sc_gather(t0, t1, x0, y0, x1, y1, b_ids, i_ids, j_ids, c0f, c1f, lc):
    m = b_ids.shape[0]
    d_f = t0.shape[1]
    d_c = c0f.shape[1]
    nlc = x0.shape[0]
    zrow = t0.shape[0] - 1

    n = nlc // lc
    whf = (t0.shape[0] - 1) // n
    hf = int(round(whf ** 0.5))
    wf = whf // hf

    ntiles = 2 * _LANES
    mpt = (2 * m) // ntiles
    nchunks = mpt // 128

    mesh = plsc.VectorSubcoreMesh(core_axis_name="c", subcore_axis_name="s")

    @functools.partial(
        pl.kernel,
        mesh=mesh,
        out_type=(
            jax.ShapeDtypeStruct((_WIN2 * m, d_f), jnp.float32),
            jax.ShapeDtypeStruct((_WIN2 * m, d_f), jnp.float32),
            jax.ShapeDtypeStruct((m, d_c), jnp.float32),
            jax.ShapeDtypeStruct((m, d_c), jnp.float32),
        ),
        scratch_types=[
            pltpu.VMEM((mpt,), jnp.int32),
            pltpu.VMEM((mpt,), jnp.int32),
            pltpu.VMEM((mpt,), jnp.int32),
            pltpu.VMEM((mpt,), jnp.int32),
            pltpu.VMEM((mpt,), jnp.int32),
            pltpu.VMEM((128,), jnp.int32),
            pltpu.VMEM((mpt, d_f), jnp.float32),
            pltpu.VMEM((128, d_c), jnp.float32),
            pltpu.SemaphoreType.DMA,
        ],
    )
    def gather_kernel(t0h, t1h, x0h, y0h, x1h, y1h, bh, ih, jh, c0h, c1h,
                      f0h, f1h, cr0h, cr1h,
                      bbuf, idbuf, pxbuf, pybuf, pbbuf,
                      idxbuf, rows, crows, sem):
        wid = lax.axis_index("s") * 2 + lax.axis_index("c")
        side = wid // _LANES
        q = wid % _LANES
        base_m = q * mpt

        def work(th, xh, yh, idsh, ch, fh, crh):
            pltpu.sync_copy(bh.at[pl.ds(base_m, mpt)], bbuf)
            pltpu.sync_copy(idsh.at[pl.ds(base_m, mpt)], idbuf)

            for c in range(nchunks):
                def cfill(t, _):
                    ssl = pl.ds(c * 128 + t * _LANES, _LANES)
                    dsl = pl.ds(t * _LANES, _LANES)
                    idxbuf[dsl] = bbuf[ssl] * lc + idbuf[ssl]
                    return 0

                lax.fori_loop(0, 128 // _LANES, cfill, 0)
                csl = pl.ds(c * 128, 128)
                pltpu.async_copy(xh.at[idxbuf], pxbuf.at[csl], sem).wait()
                pltpu.async_copy(yh.at[idxbuf], pybuf.at[csl], sem).wait()
                pltpu.async_copy(ch.at[idxbuf], crows, sem).wait()
                pltpu.sync_copy(crows, crh.at[pl.ds(base_m + c * 128, 128)])

            def bfill(t, _):
                sl = pl.ds(t * _LANES, _LANES)
                pbbuf[sl] = bbuf[sl] * whf
                return 0

            lax.fori_loop(0, mpt // _LANES, bfill, 0)

            def kstep(kk, _):
                dx = kk % _WIN - 2
                dy = kk // _WIN - 2
                for c in range(nchunks):
                    def wfill(t, _):
                        ssl = pl.ds(c * 128 + t * _LANES, _LANES)
                        dsl = pl.ds(t * _LANES, _LANES)
                        xv = pxbuf[ssl] + dx
                        yv = pybuf[ssl] + dy
                        valid = ((xv >= 0) & (xv < wf)
                                 & (yv >= 0) & (yv < hf))
                        idxbuf[dsl] = jnp.where(
                            valid, pbbuf[ssl] + xv * hf + yv, zrow)
                        return 0

                    lax.fori_loop(0, 128 // _LANES, wfill, 0)
                    pltpu.async_copy(
                        th.at[idxbuf], rows.at[pl.ds(c * 128, 128)],
                        sem).wait()
                pltpu.sync_copy(rows, fh.at[pl.ds(kk * m + base_m, mpt)])
                return 0

            lax.fori_loop(0, _WIN2, kstep, 0)

        @pl.when(side == 0)
        def _():
            work(t0h, x0h, y0h, ih, c0h, f0h, cr0h)

        @pl.when(side == 1)
        def _():
            work(t1h, x1h, y1h, jh, c1h, f1h, cr1h)

    return gather_kernel(t0, t1, x0, y0, x1, y1, b_ids, i_ids, j_ids,
                         c0f, c1f)


def _tc_merge(f3, cr, w_down, w_merge, b_down, b_merge, interpret=False):
    m, d_c = cr.shape
    d_f = w_down.shape[0]
    wm1 = w_merge[:, :d_f]
    wm2 = w_merge[:, d_f:]
    bd = b_down.reshape(1, d_f)
    bm = b_merge.reshape(1, d_f)
    blk = 128
    grid = m // blk
    hp = lax.Precision.HIGHEST
    cdims = (((1,), (1,)), ((), ()))

    def body(f_ref, c_ref, wd_ref, w1_ref, w2_ref, bd_ref, bm_ref, o_ref):
        d = lax.dot_general(c_ref[...], wd_ref[...], cdims,
                            precision=hp) + bd_ref[...]
        e = lax.dot_general(d, w2_ref[...], cdims, precision=hp) + bm_ref[...]
        for kk in range(_WIN2):
            fo = lax.dot_general(f_ref[kk], w1_ref[...], cdims, precision=hp)
            o_ref[:, kk, :] = fo + e

    return pl.pallas_call(
        body,
        grid=(grid,),
        in_specs=[
            pl.BlockSpec((_WIN2, blk, d_f), lambda g: (0, g, 0)),
            pl.BlockSpec((blk, d_c), lambda g: (g, 0)),
            pl.BlockSpec((d_f, d_c), lambda g: (0, 0)),
            pl.BlockSpec((d_f, d_f), lambda g: (0, 0)),
            pl.BlockSpec((d_f, d_f), lambda g: (0, 0)),
            pl.BlockSpec((1, d_f), lambda g: (0, 0)),
            pl.BlockSpec((1, d_f), lambda g: (0, 0)),
        ],
        out_specs=pl.BlockSpec((blk, _WIN2, d_f), lambda g: (g, 0, 0)),
        out_shape=jax.ShapeDtypeStruct((m, _WIN2, d_f), jnp.float32),
        interpret=interpret,
    )(f3, cr, w_down, wm1, wm2, bd, bm)


def kernel(feat_f0, feat_f1, feat_c0, feat_c1, coord0_f, coord1_f,
           b_ids, i_ids, j_ids, image0, W_down, b_down, W_merge, b_merge):
    n, c_f, hf, wf = feat_f0.shape
    lc = feat_c0.shape[1]
    d_c = feat_c0.shape[2]
    m = b_ids.shape[0]

    zero_row = jnp.zeros((1, c_f), jnp.float32)
    t0 = jnp.concatenate(
        [jnp.transpose(feat_f0, (0, 3, 2, 1)).reshape(n * wf * hf, c_f),
         zero_row], axis=0)
    t1 = jnp.concatenate(
        [jnp.transpose(feat_f1, (0, 3, 2, 1)).reshape(n * wf * hf, c_f),
         zero_row], axis=0)

    coord0 = jnp.round(coord0_f).astype(jnp.int32)
    coord1 = jnp.round(coord1_f).astype(jnp.int32)
    x0 = coord0[..., 0].reshape(n * lc)
    y0 = coord0[..., 1].reshape(n * lc)
    x1 = coord1[..., 0].reshape(n * lc)
    y1 = coord1[..., 1].reshape(n * lc)

    c0f = feat_c0.reshape(n * lc, d_c)
    c1f = feat_c1.reshape(n * lc, d_c)

    f0, f1, cr0, cr1 = _sc_gather(
        t0, t1, x0, y0, x1, y1,
        b_ids.astype(jnp.int32), i_ids.astype(jnp.int32),
        j_ids.astype(jnp.int32), c0f, c1f, lc)

    d_f = W_down.shape[0]
    out0 = _tc_merge(f0.reshape(_WIN2, m, d_f), cr0,
                     W_down, W_merge, b_down, b_merge)
    out1 = _tc_merge(f1.reshape(_WIN2, m, d_f), cr1,
                     W_down, W_merge, b_down, b_merge)
    return (out0, out1)


# speedup vs baseline: 1.8486x; 1.0183x over previous
import functools

import jax
import jax.numpy as jnp
from jax import lax
from jax.experimental import pallas as pl
from jax.experimental.pallas import tpu as pltpu
from jax.experimental.pallas import tpu_sc as plsc

_WIN = 5
_WIN2 = _WIN * _WIN
_LANES = 16


def _sc_gather(t0, t1, x0, y0, x1, y1, b_ids, i_ids, j_ids, c0f, c1f,
               lc, whf, hf, wf):
    m = b_ids.shape[0]
    dw = t0.shape[1]
    d_c = c0f.shape[1]
    zrow = t0.shape[0] - 1

    ntiles = 2 * _LANES
    mpt = (2 * m) // ntiles
    nchunks = mpt // 128

    mesh = plsc.VectorSubcoreMesh(core_axis_name="c", subcore_axis_name="s")

    @functools.partial(
        pl.kernel,
        mesh=mesh,
        out_type=(
            jax.ShapeDtypeStruct((_WIN2 * m, dw), jnp.float32),
            jax.ShapeDtypeStruct((_WIN2 * m, dw), jnp.float32),
            jax.ShapeDtypeStruct((m, d_c), jnp.float32),
            jax.ShapeDtypeStruct((m, d_c), jnp.float32),
        ),
        scratch_types=[
            pltpu.VMEM((mpt,), jnp.int32),
            pltpu.VMEM((mpt,), jnp.int32),
            pltpu.VMEM((mpt,), jnp.int32),
            pltpu.VMEM((mpt,), jnp.int32),
            pltpu.VMEM((mpt,), jnp.int32),
            pltpu.VMEM((128,), jnp.int32),
            pltpu.VMEM((128,), jnp.int32),
            pltpu.VMEM((mpt, dw), jnp.float32),
            pltpu.VMEM((128, d_c), jnp.float32),
            pltpu.SemaphoreType.DMA,
            pltpu.SemaphoreType.DMA,
        ],
    )
    def gather_kernel(t0h, t1h, x0h, y0h, x1h, y1h, bh, ih, jh, c0h, c1h,
                      f0h, f1h, cr0h, cr1h,
                      bbuf, idbuf, pxbuf, pybuf, pbbuf,
                      idx0, idx1, rows, crows, sem_g, sem_w):
        wid = lax.axis_index("s") * 2 + lax.axis_index("c")
        side = wid // _LANES
        q = wid % _LANES
        base_m = q * mpt
        idxbufs = [idx0, idx1]

        def work(th, xh, yh, idsh, ch, fh, crh):
            pltpu.sync_copy(bh.at[pl.ds(base_m, mpt)], bbuf)
            pltpu.sync_copy(idsh.at[pl.ds(base_m, mpt)], idbuf)

            for c in range(nchunks):
                ib = idxbufs[c % 2]

                def cfill(t, _):
                    ssl = pl.ds(c * 128 + t * _LANES, _LANES)
                    dsl = pl.ds(t * _LANES, _LANES)
                    ib[dsl] = bbuf[ssl] * lc + idbuf[ssl]
                    return 0

                lax.fori_loop(0, 128 // _LANES, cfill, 0)
                csl = pl.ds(c * 128, 128)
                g0 = pltpu.async_copy(xh.at[ib], pxbuf.at[csl], sem_g)
                g1 = pltpu.async_copy(yh.at[ib], pybuf.at[csl], sem_g)
                g2 = pltpu.async_copy(ch.at[ib], crows, sem_g)
                g0.wait()
                g1.wait()
                g2.wait()
                pltpu.sync_copy(crows, crh.at[pl.ds(base_m + c * 128, 128)])

            def bfill(t, _):
                sl = pl.ds(t * _LANES, _LANES)
                pbbuf[sl] = bbuf[sl] * whf
                return 0

            lax.fori_loop(0, mpt // _LANES, bfill, 0)

            def kbody(kk, first):
                dx = kk % _WIN - 2
                dy = kk // _WIN - 2
                gathers = []
                for c in range(nchunks):
                    ib = idxbufs[c % 2]

                    def wfill(t, _):
                        ssl = pl.ds(c * 128 + t * _LANES, _LANES)
                        dsl = pl.ds(t * _LANES, _LANES)
                        xv = pxbuf[ssl] + dx
                        yv = pybuf[ssl] + dy
                        valid = ((xv >= 0) & (xv < wf)
                                 & (yv >= 0) & (yv < hf))
                        ib[dsl] = jnp.where(
                            valid, pbbuf[ssl] + xv * hf + yv, zrow)
                        return 0

                    lax.fori_loop(0, 128 // _LANES, wfill, 0)
                    if c == 0 and not first:
                        pltpu.make_async_copy(
                            rows, fh.at[pl.ds(0, mpt)], sem_w).wait()
                    gathers.append(pltpu.async_copy(
                        th.at[ib], rows.at[pl.ds(c * 128, 128)], sem_g))
                for g in gathers:
                    g.wait()
                pltpu.async_copy(
                    rows, fh.at[pl.ds(kk * m + base_m, mpt)], sem_w)

            kbody(0, True)

            def kstep(kk, _):
                kbody(kk, False)
                return 0

            lax.fori_loop(1, _WIN2, kstep, 0)
            pltpu.make_async_copy(rows, fh.at[pl.ds(0, mpt)], sem_w).wait()

        @pl.when(side == 0)
        def _():
            work(t0h, x0h, y0h, ih, c0h, f0h, cr0h)

        @pl.when(side == 1)
        def _():
            work(t1h, x1h, y1h, jh, c1h, f1h, cr1h)

    return gather_kernel(t0, t1, x0, y0, x1, y1, b_ids, i_ids, j_ids,
                         c0f, c1f)


def _tc_merge(f3, cr, w_down, w_merge, b_down, b_merge, interpret=False):
    m, d_c = cr.shape
    d_f = w_down.shape[0]
    wm1 = w_merge[:, :d_f]
    wm2 = w_merge[:, d_f:]
    bd = b_down.reshape(1, d_f)
    bm = b_merge.reshape(1, d_f)
    blk = 128
    grid = m // blk
    hp = lax.Precision.HIGHEST
    cdims = (((1,), (1,)), ((), ()))

    def body(f_ref, c_ref, wd_ref, w1_ref, w2_ref, bd_ref, bm_ref, o_ref):
        d = lax.dot_general(c_ref[...], wd_ref[...], cdims,
                            precision=hp) + bd_ref[...]
        e = lax.dot_general(d, w2_ref[...], cdims, precision=hp) + bm_ref[...]
        for kk in range(_WIN2):
            fo = lax.dot_general(f_ref[kk], w1_ref[...], cdims, precision=hp)
            o_ref[:, kk, :] = fo + e

    return pl.pallas_call(
        body,
        grid=(grid,),
        in_specs=[
            pl.BlockSpec((_WIN2, blk, d_f), lambda g: (0, g, 0)),

            pl.BlockSpec((blk, d_c), lambda g: (g, 0)),
            pl.BlockSpec((d_f, d_c), lambda g: (0, 0)),
            pl.BlockSpec((d_f, d_f), lambda g: (0, 0)),
            pl.BlockSpec((d_f, d_f), lambda g: (0, 0)),
            pl.BlockSpec((1, d_f), lambda g: (0, 0)),
            pl.BlockSpec((1, d_f), lambda g: (0, 0)),
        ],
        out_specs=pl.BlockSpec((blk, _WIN2, d_f), lambda g: (g, 0, 0)),
        out_shape=jax.ShapeDtypeStruct((m, _WIN2, d_f), jnp.float32),
        interpret=interpret,
    )(f3, cr, w_down, wm1, wm2, bd, bm)


def kernel(feat_f0, feat_f1, feat_c0, feat_c1, coord0_f, coord1_f,
           b_ids, i_ids, j_ids, image0, W_down, b_down, W_merge, b_merge):
    n, c_f, hf, wf = feat_f0.shape
    lc = feat_c0.shape[1]
    d_c = feat_c0.shape[2]
    m = b_ids.shape[0]

    zero_row = jnp.zeros((1, c_f), jnp.float32)
    t0 = jnp.concatenate(
        [jnp.transpose(feat_f0, (0, 3, 2, 1)).reshape(n * wf * hf, c_f),
         zero_row], axis=0)
    t1 = jnp.concatenate(
        [jnp.transpose(feat_f1, (0, 3, 2, 1)).reshape(n * wf * hf, c_f),
         zero_row], axis=0)

    coord0 = jnp.round(coord0_f).astype(jnp.int32)
    coord1 = jnp.round(coord1_f).astype(jnp.int32)
    x0 = coord0[..., 0].reshape(n * lc)
    y0 = coord0[..., 1].reshape(n * lc)
    x1 = coord1[..., 0].reshape(n * lc)
    y1 = coord1[..., 1].reshape(n * lc)

    c0f = feat_c0.reshape(n * lc, d_c)
    c1f = feat_c1.reshape(n * lc, d_c)

    f0, f1, cr0, cr1 = _sc_gather(
        t0, t1, x0, y0, x1, y1,
        b_ids.astype(jnp.int32), i_ids.astype(jnp.int32),
        j_ids.astype(jnp.int32), c0f, c1f, lc, wf * hf, hf, wf)

    d_f = W_down.shape[0]
    out0 = _tc_merge(f0.reshape(_WIN2, m, d_f), cr0,
                     W_down, W_merge, b_down, b_merge)
    out1 = _tc_merge(f1.reshape(_WIN2, m, d_f), cr1,
                     W_down, W_merge, b_down, b_merge)
    return (out0, out1)
